# baseline (speedup 1.0000x reference)
import functools
import jax
import jax.numpy as jnp
from jax import lax
from jax.experimental import pallas as pl
from jax.experimental.pallas import tpu as pltpu

B = 1024
N = 100000
D_IN = 64
D_MAIN = 128
D_BLOCK = 256
C = 96
EPS = 1e-5

N_PAD = 100352
ENC_BLK = 2048
SCORE_QB = 128
ATT_QB = 64
NEG = -1e30


def _encode_cand_kernel(x_ref, wl_ref, bl_ref, wk_ref, bk_ref, ck_ref):
    h = jnp.dot(x_ref[...], wl_ref[...], preferred_element_type=jnp.float32)
    h = h + bl_ref[...]
    ck = jnp.dot(h, wk_ref[...], preferred_element_type=jnp.float32)
    ck_ref[...] = ck + bk_ref[...]


def _encode_query_kernel(x_ref, wl_ref, bl_ref, wk_ref, bk_ref, xq_ref, kq_ref):
    h = jnp.dot(x_ref[...], wl_ref[...], preferred_element_type=jnp.float32)
    h = h + bl_ref[...]
    xq_ref[...] = h
    kq_ref[...] = jnp.dot(h, wk_ref[...], preferred_element_type=jnp.float32) + bk_ref[...]


def _scores_kernel(kq_ref, ck_ref, s_ref):
    j = pl.program_id(1)
    ck = ck_ref[...]
    nsq = jnp.sum(ck * ck, axis=1)[None, :]
    s = 2.0 * jax.lax.dot_general(
        kq_ref[...], ck, (((1,), (1,)), ((), ())),
        preferred_element_type=jnp.float32) - nsq
    col = j * ENC_BLK + lax.broadcasted_iota(jnp.int32, s.shape, 1)
    s_ref[...] = jnp.where(col < N, s, NEG)


def _attn_kernel(xq_ref, kq_ref, ctxk_ref, ctxy_ref,
                 wlab_ref, blab_ref, wt1_ref, bt1_ref, wt2_ref,
                 ln1g_ref, ln1b_ref, wb1_ref, bb1_ref, wb2_ref, bb2_ref,
                 lnhg_ref, lnhb_ref, whead_ref, bhead_ref, out_ref):
    nq = xq_ref.shape[0]
    kq = kq_ref[...]
    ctxk = ctxk_ref[...]
    kq_rows = jnp.repeat(kq, C, axis=0)
    diff = kq_rows - ctxk

    sim = -jnp.sum(diff * diff, axis=1, keepdims=True)
    sim3 = sim.reshape(nq, C, 1)
    m = jnp.max(sim3, axis=1, keepdims=True)
    e = jnp.exp(sim3 - m)
    denom = jnp.sum(e, axis=1, keepdims=True)
    probs = e / denom

    y_emb = ctxy_ref[...] * wlab_ref[...] + blab_ref[...]

    t = jnp.dot(diff, wt1_ref[...], preferred_element_type=jnp.float32) + bt1_ref[...]
    t = jnp.maximum(t, 0.0)
    t = jnp.dot(t, wt2_ref[...], preferred_element_type=jnp.float32)

    values = (y_emb + t).reshape(nq, C, D_MAIN)
    ctx_x = jnp.sum(values * probs, axis=1)

    x = xq_ref[...] + ctx_x

    mu = jnp.mean(x, axis=1, keepdims=True)
    var = jnp.mean((x - mu) ** 2, axis=1, keepdims=True)
    h = (x - mu) * lax.rsqrt(var + EPS) * ln1g_ref[...] + ln1b_ref[...]
    h = jnp.dot(h, wb1_ref[...], preferred_element_type=jnp.float32) + bb1_ref[...]
    h = jnp.maximum(h, 0.0)
    x = x + jnp.dot(h, wb2_ref[...], preferred_element_type=jnp.float32) + bb2_ref[...]

    mu = jnp.mean(x, axis=1, keepdims=True)
    var = jnp.mean((x - mu) ** 2, axis=1, keepdims=True)
    h = (x - mu) * lax.rsqrt(var + EPS) * lnhg_ref[...] + lnhb_ref[...]
    h = jnp.maximum(h, 0.0)
    out_ref[...] = jnp.dot(h, whead_ref[...], preferred_element_type=jnp.float32) + bhead_ref[...]


def _full(shape):
    return pl.BlockSpec(shape, lambda *_: tuple(0 for _ in shape))


def kernel(x_num, candidate_x_num, candidate_y, W_lin, b_lin, W_K, b_K,
           W_label, b_label, W_T1, b_T1, W_T2, ln1_g, ln1_b,
           W_b1, b_b1, W_b2, b_b2, lnh_g, lnh_b, W_head, b_head):
    f32 = jnp.float32

    cand_pad = jnp.pad(candidate_x_num, ((0, N_PAD - N), (0, 0)))
    ck = pl.pallas_call(
        _encode_cand_kernel,
        grid=(N_PAD // ENC_BLK,),
        in_specs=[
            pl.BlockSpec((ENC_BLK, D_IN), lambda i: (i, 0)),
            _full((D_IN, D_MAIN)),
            _full((D_MAIN,)),
            _full((D_MAIN, D_MAIN)),
            _full((D_MAIN,)),
        ],
        out_specs=pl.BlockSpec((ENC_BLK, D_MAIN), lambda i: (i, 0)),
        out_shape=jax.ShapeDtypeStruct((N_PAD, D_MAIN), f32),
    )(cand_pad, W_lin, b_lin, W_K, b_K)

    xq, kq = pl.pallas_call(
        _encode_query_kernel,
        grid=(1,),
        in_specs=[
            _full((B, D_IN)),
            _full((D_IN, D_MAIN)),
            _full((D_MAIN,)),
            _full((D_MAIN, D_MAIN)),
            _full((D_MAIN,)),
        ],
        out_specs=[_full((B, D_MAIN)), _full((B, D_MAIN))],
        out_shape=[jax.ShapeDtypeStruct((B, D_MAIN), f32),
                   jax.ShapeDtypeStruct((B, D_MAIN), f32)],
    )(x_num, W_lin, b_lin, W_K, b_K)

    scores = pl.pallas_call(
        _scores_kernel,
        grid=(B // SCORE_QB, N_PAD // ENC_BLK),
        in_specs=[
            pl.BlockSpec((SCORE_QB, D_MAIN), lambda i, j: (i, 0)),
            pl.BlockSpec((ENC_BLK, D_MAIN), lambda i, j: (j, 0)),
        ],
        out_specs=pl.BlockSpec((SCORE_QB, ENC_BLK), lambda i, j: (i, j)),
        out_shape=jax.ShapeDtypeStruct((B, N_PAD), f32),
    )(kq, ck)

    _, context_idx = lax.top_k(scores, C)

    idx_flat = context_idx.reshape(-1)
    ctx_k = ck[idx_flat]
    ctx_y = jnp.pad(candidate_y, (0, N_PAD - N))[idx_flat][:, None]

    W_head_pad = jnp.pad(W_head, ((0, 0), (0, D_MAIN - 2)))
    b_head_pad = jnp.pad(b_head, (0, D_MAIN - 2))
    out = pl.pallas_call(
        _attn_kernel,
        grid=(B // ATT_QB,),
        in_specs=[
            pl.BlockSpec((ATT_QB, D_MAIN), lambda i: (i, 0)),
            pl.BlockSpec((ATT_QB, D_MAIN), lambda i: (i, 0)),
            pl.BlockSpec((ATT_QB * C, D_MAIN), lambda i: (i, 0)),
            pl.BlockSpec((ATT_QB * C, 1), lambda i: (i, 0)),
            _full((1, D_MAIN)),
            _full((D_MAIN,)),
            _full((D_MAIN, D_BLOCK)),
            _full((D_BLOCK,)),
            _full((D_BLOCK, D_MAIN)),
            _full((D_MAIN,)),
            _full((D_MAIN,)),
            _full((D_MAIN, D_BLOCK)),
            _full((D_BLOCK,)),
            _full((D_BLOCK, D_MAIN)),
            _full((D_MAIN,)),
            _full((D_MAIN,)),
            _full((D_MAIN,)),
            _full((D_MAIN, D_MAIN)),
            _full((D_MAIN,)),
        ],
        out_specs=pl.BlockSpec((ATT_QB, D_MAIN), lambda i: (i, 0)),
        out_shape=jax.ShapeDtypeStruct((B, D_MAIN), f32),
    )(xq, kq, ctx_k, ctx_y, W_label, b_label, W_T1, b_T1, W_T2,
      ln1_g, ln1_b, W_b1, b_b1, W_b2, b_b2, lnh_g, lnh_b,
      W_head_pad, b_head_pad)

    return out[:, :2]
